# baseline (speedup 1.0000x reference)
import jax
import jax.numpy as jnp
from jax import lax
from jax.experimental import pallas as pl
from jax.experimental.pallas import tpu as pltpu
from jax.experimental.pallas import tpu_sc as plsc

_NC = 2
_NS = 16
_NW = _NC * _NS
_LANES = 16

_B = 4096
_E = 64
_H_PAD = 56
_CH = 2
_IDX_PER_DMA = _CH * _H_PAD
_B_PER_W = _B // _NW
_CHUNKS = _B_PER_W // _CH


def _gather_sum_body(idx_hbm, table_hbm, out_hbm, idx_v, rows_v, outb_v, sem):
    wid = lax.axis_index("s") * _NC + lax.axis_index("c")
    base = wid * _B_PER_W

    ioff = pl.multiple_of(base * _H_PAD, 8)
    pltpu.sync_copy(idx_hbm.at[pl.ds(ioff, _B_PER_W * _H_PAD)], idx_v)

    def chunk_body(c, carry):
        off = pl.multiple_of(c * _IDX_PER_DMA, 8)
        pltpu.async_copy(
            table_hbm.at[idx_v.at[pl.ds(off, _IDX_PER_DMA)]], rows_v, sem
        ).wait()
        for e in range(_CH):
            acc = [jnp.zeros((_LANES,), jnp.float32) for _ in range(_E // _LANES)]
            for j in range(_H_PAD):
                r = e * _H_PAD + j
                for q in range(_E // _LANES):
                    acc[q] = acc[q] + rows_v[r, pl.ds(q * _LANES, _LANES)]
            row = c * _CH + e
            for q in range(_E // _LANES):
                outb_v[row, pl.ds(q * _LANES, _LANES)] = acc[q]
        return carry

    lax.fori_loop(0, _CHUNKS, chunk_body, 0)

    pltpu.sync_copy(outb_v, out_hbm.at[pl.ds(pl.multiple_of(base, 8), _B_PER_W)])


def _gather_sum(idx_flat, table):
    k = pl.kernel(
        _gather_sum_body,
        out_type=jax.ShapeDtypeStruct((_B, _E), jnp.float32),
        mesh=plsc.VectorSubcoreMesh(
            core_axis_name="c", subcore_axis_name="s",
            num_cores=_NC, num_subcores=_NS,
        ),
        scratch_types=[
            pltpu.VMEM((_B_PER_W * _H_PAD,), jnp.int32),
            pltpu.VMEM((_IDX_PER_DMA, _E), jnp.float32),
            pltpu.VMEM((_B_PER_W, _E), jnp.float32),
            pltpu.SemaphoreType.DMA,
        ],
        compiler_params=pltpu.CompilerParams(use_tc_tiling_on_sc=False),
    )
    return k(idx_flat, table)


def _proj_body(x_ref, w_ref, b_ref, o_ref):
    x = jnp.maximum(x_ref[...], 0.0)
    o_ref[...] = (
        lax.dot_general(
            x, w_ref[...],
            dimension_numbers=(((1,), (1,)), ((), ())),
            preferred_element_type=jnp.float32,
        )
        + b_ref[...]
    )


def _proj(x, W, b2d):
    B, E = x.shape
    T = W.shape[0]
    blk = 512
    return pl.pallas_call(
        _proj_body,
        grid=(B // blk,),
        in_specs=[
            pl.BlockSpec((blk, E), lambda i: (i, 0)),
            pl.BlockSpec((T, E), lambda i: (0, 0)),
            pl.BlockSpec((1, T), lambda i: (0, 0)),
        ],
        out_specs=pl.BlockSpec((blk, T), lambda i: (i, 0)),
        out_shape=jax.ShapeDtypeStruct((B, T), jnp.float32),
    )(x, W, b2d)


def kernel(input_text, table, W, b):
    B = input_text.shape[0]
    idx = input_text.reshape(B, -1)
    pad = jnp.zeros((B, _H_PAD - idx.shape[1]), jnp.int32)
    idx_flat = jnp.concatenate([idx, pad], axis=1).reshape(-1)
    sums = _gather_sum(idx_flat, table)
    return _proj(sums, W, b.reshape(1, -1))


# speedup vs baseline: 1.6948x; 1.6948x over previous
import jax
import jax.numpy as jnp
from jax import lax
from jax.experimental import pallas as pl
from jax.experimental.pallas import tpu as pltpu
from jax.experimental.pallas import tpu_sc as plsc

_NC = 2
_NS = 16
_NW = _NC * _NS
_LANES = 16

_B = 4096
_E = 64
_H = 50
_CH = 2
_IDX_ROW = _CH * _H
_B_PER_W = _B // _NW
_CHUNKS = _B_PER_W // _CH
_NBUF = 4
_QS = _E // _LANES


def _gather_sum_body(idx_hbm, table_hbm, out_hbm,
                     idx_v, rows_v, outb_v, s0, s1, s2, s3):
    sems = (s0, s1, s2, s3)
    wid = lax.axis_index("s") * _NC + lax.axis_index("c")
    rbase = pl.multiple_of(wid * _CHUNKS, 8)
    base = pl.multiple_of(wid * _B_PER_W, 8)

    pltpu.sync_copy(idx_hbm.at[pl.ds(rbase, _CHUNKS)], idx_v)

    def gather_start(c, b):
        pltpu.async_copy(table_hbm.at[idx_v.at[c]], rows_v.at[b], sems[b])

    def gather_wait(c, b):
        pltpu.make_async_copy(
            table_hbm.at[idx_v.at[c]], rows_v.at[b], sems[b]
        ).wait()

    for b in range(_NBUF):
        gather_start(b, b)

    def reduce_elem(rb, e):
        zero = jnp.zeros((_LANES,), jnp.float32)
        init = (zero, zero, zero, zero)

        def red(j, acc):
            r = e * _H + j
            return tuple(
                acc[q] + rb[r, pl.ds(q * _LANES, _LANES)] for q in range(_QS)
            )

        return plsc.parallel_loop(0, _H, unroll=10, carry=init)(red)

    def t_body(t, carry):
        for b in range(_NBUF):
            c = t * _NBUF + b
            gather_wait(c, b)
            rb = rows_v.at[b]
            for e in range(_CH):
                acc = reduce_elem(rb, e)
                row = c * _CH + e
                for q in range(_QS):
                    outb_v[row, pl.ds(q * _LANES, _LANES)] = acc[q]
            nc = c + _NBUF

            @pl.when(nc < _CHUNKS)
            def _():
                gather_start(nc, b)

        return carry

    lax.fori_loop(0, _CHUNKS // _NBUF, t_body, 0)

    pltpu.sync_copy(outb_v, out_hbm.at[pl.ds(base, _B_PER_W)])


def _gather_sum(idx2, table):
    k = pl.kernel(
        _gather_sum_body,
        out_type=jax.ShapeDtypeStruct((_B, _E), jnp.float32),
        mesh=plsc.VectorSubcoreMesh(
            core_axis_name="c", subcore_axis_name="s",
            num_cores=_NC, num_subcores=_NS,
        ),
        scratch_types=[
            pltpu.VMEM((_CHUNKS, _IDX_ROW), jnp.int32),
            pltpu.VMEM((_NBUF, _IDX_ROW, _E), jnp.float32),
            pltpu.VMEM((_B_PER_W, _E), jnp.float32),
            pltpu.SemaphoreType.DMA,
            pltpu.SemaphoreType.DMA,
            pltpu.SemaphoreType.DMA,
            pltpu.SemaphoreType.DMA,
        ],
        compiler_params=pltpu.CompilerParams(use_tc_tiling_on_sc=False),
    )
    return k(idx2, table)


def _proj_body(x_ref, w_ref, b_ref, o_ref):
    x = jnp.maximum(x_ref[...], 0.0)
    o_ref[...] = (
        lax.dot_general(
            x, w_ref[...],
            dimension_numbers=(((1,), (1,)), ((), ())),
            preferred_element_type=jnp.float32,
        )
        + b_ref[...]
    )


def _proj(x, W, b2d):
    B, E = x.shape
    T = W.shape[0]
    blk = 512
    return pl.pallas_call(
        _proj_body,
        grid=(B // blk,),
        in_specs=[
            pl.BlockSpec((blk, E), lambda i: (i, 0)),
            pl.BlockSpec((T, E), lambda i: (0, 0)),
            pl.BlockSpec((1, T), lambda i: (0, 0)),
        ],
        out_specs=pl.BlockSpec((blk, T), lambda i: (i, 0)),
        out_shape=jax.ShapeDtypeStruct((B, T), jnp.float32),
    )(x, W, b2d)


def kernel(input_text, table, W, b):
    B = input_text.shape[0]
    idx2 = input_text.reshape(B // _CH, _IDX_ROW)
    sums = _gather_sum(idx2, table)
    return _proj(sums, W, b.reshape(1, -1))
